# Initial kernel scaffold; baseline (speedup 1.0000x reference)
import jax
import jax.numpy as jnp
from jax.experimental import pallas as pl


def kernel(query_indices, visual_indices, target_indices, self_attentions, topk_query_indices):
    raise NotImplementedError("write your pallas kernel here")


# speedup vs baseline: 1.0336x; 1.0336x over previous
import functools

import jax
import jax.numpy as jnp
from jax import lax
from jax.experimental import pallas as pl
from jax.experimental.pallas import tpu as pltpu
from jax.experimental.pallas import tpu_sc as plsc

LANES = 16
NSUB = 16
NVIS = 576
NCH = NVIS // LANES
LH = 64
LH_PER_W = LH // NSUB
ROWS_PER_LH = 8
ROWS_PER_W = LH_PER_W * ROWS_PER_LH
EPS = 1e-8


def _sc_body(a_hbm, rowids_hbm, vis_hbm, out_hbm,
             idx_v, vis_v, rows_v, t_v, acc_v, hwl_v, part_v, partl_v,
             outbuf_v, hw_sh, part_sh, sem):
    w = lax.axis_index("s")
    lane = lax.iota(jnp.int32, LANES)

    base_ids = pl.multiple_of(w * ROWS_PER_W, ROWS_PER_W)
    pltpu.sync_copy(rowids_hbm.at[pl.ds(base_ids, ROWS_PER_W)], idx_v)
    pltpu.sync_copy(vis_hbm, vis_v)
    pltpu.async_copy(a_hbm.at[idx_v], rows_v, sem).wait()

    for ll in range(LH_PER_W):
        def p1_body(j, acc, ll=ll):
            cbase = pl.multiple_of(j * LANES, LANES)
            cols = vis_v[pl.ds(cbase, LANES)]
            s = jnp.zeros((LANES,), jnp.float32)
            for r in range(5):
                ridx = jnp.full((LANES,), ll * ROWS_PER_LH + r, jnp.int32)
                s = s + plsc.load_gather(rows_v, [ridx, cols])
            tidx = jnp.full((LANES,), ll * ROWS_PER_LH + 5, jnp.int32)
            t_v[ll, pl.ds(cbase, LANES)] = plsc.load_gather(rows_v, [tidx, cols])
            return acc + s
        acc = lax.fori_loop(0, NCH, p1_body, jnp.zeros((LANES,), jnp.float32))
        acc_v[ll, :] = acc

    pltpu.sync_copy(acc_v, hw_sh.at[pl.ds(pl.multiple_of(w * LH_PER_W, LH_PER_W), LH_PER_W)])
    plsc.subcore_barrier()

    pltpu.sync_copy(hw_sh, hwl_v)
    hw = []
    for jj in range(LH // LANES):
        s = jnp.zeros((LANES,), jnp.float32)
        ridx = lane + jj * LANES
        for k in range(LANES):
            cidx = jnp.full((LANES,), k, jnp.int32)
            s = s + plsc.load_gather(hwl_v, [ridx, cidx])
        hw.append(s)
    m = jnp.max(jnp.maximum(jnp.maximum(hw[0], hw[1]), jnp.maximum(hw[2], hw[3])))
    e = [jnp.exp(h - m) for h in hw]
    ssum = jnp.sum(e[0] + e[1] + e[2] + e[3])
    wv = [x / ssum for x in e]

    lh0 = w * LH_PER_W
    jjt = lh0 // LANES
    wsel = jnp.where(jjt == 0, wv[0],
                     jnp.where(jjt == 1, wv[1],
                               jnp.where(jjt == 2, wv[2], wv[3])))
    wscal = [jnp.sum(jnp.where(lane == (lh0 % LANES) + ll, wsel, 0.0))
             for ll in range(LH_PER_W)]

    def p2_body(j, c):
        cbase = pl.multiple_of(j * LANES, LANES)
        p = jnp.zeros((LANES,), jnp.float32)
        for ll in range(LH_PER_W):
            p = p + wscal[ll] * t_v[ll, pl.ds(cbase, LANES)]
        part_v[pl.ds(cbase, LANES)] = p
        return c
    lax.fori_loop(0, NCH, p2_body, 0)
    pltpu.sync_copy(part_v, part_sh.at[w])
    plsc.subcore_barrier()

    @pl.when(w == 0)
    def _():
        pltpu.sync_copy(part_sh, partl_v)

        def sum_body(j, tot):
            cbase = pl.multiple_of(j * LANES, LANES)
            s = jnp.zeros((LANES,), jnp.float32)
            for ww in range(NSUB):
                s = s + partl_v[ww, pl.ds(cbase, LANES)]
            outbuf_v[pl.ds(cbase, LANES)] = s
            return tot + s
        tot = lax.fori_loop(0, NCH, sum_body, jnp.zeros((LANES,), jnp.float32))
        denom = jnp.sum(tot) + EPS

        def norm_body(j, c):
            cbase = pl.multiple_of(j * LANES, LANES)
            outbuf_v[pl.ds(cbase, LANES)] = outbuf_v[pl.ds(cbase, LANES)] / denom
            return c
        lax.fori_loop(0, NCH, norm_body, 0)
        pltpu.sync_copy(outbuf_v, out_hbm)


def kernel(query_indices, visual_indices, target_indices, self_attentions,
           topk_query_indices):
    del query_indices, target_indices
    L, B, H, Q, S = self_attentions.shape
    a2d = self_attentions.reshape(L * B * H * Q, S)

    rows = jnp.concatenate([
        topk_query_indices.astype(jnp.int32),
        jnp.full((3,), Q - 1, jnp.int32),
    ])
    row_ids = (jnp.arange(L * H, dtype=jnp.int32) * Q)[:, None] + rows[None, :]
    row_ids = row_ids.reshape(-1)
    vis = visual_indices.astype(jnp.int32)

    mesh = plsc.VectorSubcoreMesh(core_axis_name="c", subcore_axis_name="s",
                                  num_cores=1, num_subcores=NSUB)
    run = functools.partial(
        pl.kernel,
        out_type=jax.ShapeDtypeStruct((NVIS,), jnp.float32),
        mesh=mesh,
        scratch_types=[
            pltpu.VMEM((ROWS_PER_W,), jnp.int32),
            pltpu.VMEM((NVIS,), jnp.int32),
            pltpu.VMEM((ROWS_PER_W, S), jnp.float32),
            pltpu.VMEM((LH_PER_W, NVIS), jnp.float32),
            pltpu.VMEM((LH_PER_W, LANES), jnp.float32),
            pltpu.VMEM((LH, LANES), jnp.float32),
            pltpu.VMEM((NVIS,), jnp.float32),
            pltpu.VMEM((NSUB, NVIS), jnp.float32),
            pltpu.VMEM((NVIS,), jnp.float32),
            pltpu.VMEM_SHARED((LH, LANES), jnp.float32),
            pltpu.VMEM_SHARED((NSUB, NVIS), jnp.float32),
            pltpu.SemaphoreType.DMA,
        ],
        compiler_params=pltpu.CompilerParams(use_tc_tiling_on_sc=False,
                                             needs_layout_passes=False),
    )(_sc_body)
    out = run(a2d, row_ids, vis)
    return out.reshape(1, NVIS)


# speedup vs baseline: 26.1608x; 25.3103x over previous
import functools

import jax
import jax.numpy as jnp
from jax import lax
from jax.experimental import pallas as pl
from jax.experimental.pallas import tpu as pltpu
from jax.experimental.pallas import tpu_sc as plsc

LANES = 16
NSUB = 16
NVIS = 576
NCH = NVIS // LANES
LH = 64
LH_PER_W = LH // NSUB
ROWS_PER_LH = 8
ROWS_PER_W = LH_PER_W * ROWS_PER_LH
EPS = 1e-8


def _sc_body(a_hbm, rowids_hbm, vis_hbm, out_hbm,
             idx_v, vis_v, rows_v, t_v, acc_v, hwl_v, part_v, partl_v,
             outbuf_v, hw_sh, part_sh, sem):
    w = lax.axis_index("s")
    lane = lax.iota(jnp.int32, LANES)

    base_ids = pl.multiple_of(w * ROWS_PER_W, ROWS_PER_W)
    pltpu.sync_copy(rowids_hbm.at[pl.ds(base_ids, ROWS_PER_W)], idx_v)
    pltpu.sync_copy(vis_hbm, vis_v)
    pltpu.async_copy(a_hbm.at[idx_v], rows_v, sem).wait()

    for ll in range(LH_PER_W):
        def p1_body(j, acc, ll=ll):
            cbase = pl.multiple_of(j * LANES, LANES)
            cols = vis_v[pl.ds(cbase, LANES)]
            s = jnp.zeros((LANES,), jnp.float32)
            for r in range(5):
                ridx = jnp.full((LANES,), ll * ROWS_PER_LH + r, jnp.int32)
                s = s + plsc.load_gather(rows_v, [ridx, cols])
            tidx = jnp.full((LANES,), ll * ROWS_PER_LH + 5, jnp.int32)
            t_v[ll, pl.ds(cbase, LANES)] = plsc.load_gather(rows_v, [tidx, cols])
            return acc + s
        acc = lax.fori_loop(0, NCH, p1_body, jnp.zeros((LANES,), jnp.float32))
        acc_v[ll, :] = acc

    pltpu.sync_copy(acc_v, hw_sh.at[pl.ds(pl.multiple_of(w * LH_PER_W, LH_PER_W), LH_PER_W)])
    plsc.subcore_barrier()

    pltpu.sync_copy(hw_sh, hwl_v)
    hw = []
    for jj in range(LH // LANES):
        s = jnp.zeros((LANES,), jnp.float32)
        ridx = lane + jj * LANES
        for k in range(LANES):
            cidx = jnp.full((LANES,), k, jnp.int32)
            s = s + plsc.load_gather(hwl_v, [ridx, cidx])
        hw.append(s)
    m = jnp.max(jnp.maximum(jnp.maximum(hw[0], hw[1]), jnp.maximum(hw[2], hw[3])))
    e = [jnp.exp(h - m) for h in hw]
    ssum = jnp.sum(e[0] + e[1] + e[2] + e[3])
    wv = [x / ssum for x in e]

    lh0 = w * LH_PER_W
    jjt = lh0 // LANES
    wsel = jnp.where(jjt == 0, wv[0],
                     jnp.where(jjt == 1, wv[1],
                               jnp.where(jjt == 2, wv[2], wv[3])))
    wscal = [jnp.sum(jnp.where(lane == (lh0 % LANES) + ll, wsel, 0.0))
             for ll in range(LH_PER_W)]

    def p2_body(j, c):
        cbase = pl.multiple_of(j * LANES, LANES)
        p = jnp.zeros((LANES,), jnp.float32)
        for ll in range(LH_PER_W):
            p = p + wscal[ll] * t_v[ll, pl.ds(cbase, LANES)]
        part_v[pl.ds(cbase, LANES)] = p
        return c
    lax.fori_loop(0, NCH, p2_body, 0)
    pltpu.sync_copy(part_v, part_sh.at[w])
    plsc.subcore_barrier()

    @pl.when(w == 0)
    def _():
        pltpu.sync_copy(part_sh, partl_v)

        def sum_body(j, tot):
            cbase = pl.multiple_of(j * LANES, LANES)
            s = jnp.zeros((LANES,), jnp.float32)
            for ww in range(NSUB):
                s = s + partl_v[ww, pl.ds(cbase, LANES)]
            outbuf_v[pl.ds(cbase, LANES)] = s
            return tot + s
        tot = lax.fori_loop(0, NCH, sum_body, jnp.zeros((LANES,), jnp.float32))
        denom = jnp.sum(tot) + EPS

        def norm_body(j, c):
            cbase = pl.multiple_of(j * LANES, LANES)
            outbuf_v[pl.ds(cbase, LANES)] = outbuf_v[pl.ds(cbase, LANES)] / denom
            return c
        lax.fori_loop(0, NCH, norm_body, 0)
        pltpu.sync_copy(outbuf_v, out_hbm)


def kernel(query_indices, visual_indices, target_indices, self_attentions,
           topk_query_indices):
    del query_indices, target_indices
    L, B, H, Q, S = self_attentions.shape
    a2d = self_attentions.reshape(L * B * H * Q, S)

    rows = jnp.concatenate([
        topk_query_indices.astype(jnp.int32),
        jnp.full((3,), Q - 1, jnp.int32),
    ])
    row_ids = (jnp.arange(L * H, dtype=jnp.int32) * Q)[:, None] + rows[None, :]
    row_ids = row_ids.reshape(-1)
    vis = visual_indices.astype(jnp.int32)

    mesh = plsc.VectorSubcoreMesh(core_axis_name="c", subcore_axis_name="s",
                                  num_cores=1, num_subcores=NSUB)
    run = functools.partial(
        pl.kernel,
        out_type=jax.ShapeDtypeStruct((NVIS,), jnp.float32),
        mesh=mesh,
        scratch_types=[
            pltpu.VMEM((ROWS_PER_W,), jnp.int32),
            pltpu.VMEM((NVIS,), jnp.int32),
            pltpu.VMEM((ROWS_PER_W, S), jnp.float32),
            pltpu.VMEM((LH_PER_W, NVIS), jnp.float32),
            pltpu.VMEM((LH_PER_W, LANES), jnp.float32),
            pltpu.VMEM((LH, LANES), jnp.float32),
            pltpu.VMEM((NVIS,), jnp.float32),
            pltpu.VMEM((NSUB, NVIS), jnp.float32),
            pltpu.VMEM((NVIS,), jnp.float32),
            pltpu.VMEM_SHARED((LH, LANES), jnp.float32),
            pltpu.VMEM_SHARED((NSUB, NVIS), jnp.float32),
            pltpu.SemaphoreType.DMA,
        ],
        compiler_params=pltpu.CompilerParams(needs_layout_passes=False),
    )(_sc_body)
    out = run(a2d, row_ids, vis)
    return out.reshape(1, NVIS)
